# Initial kernel scaffold; baseline (speedup 1.0000x reference)
import functools

import jax
import jax.numpy as jnp
from jax.experimental import pallas as pl
from jax.experimental.pallas import tpu as pltpu

_N = 8192
_BR = 256


def _linear_sq_kernel(x_ref, w_ref, y_ref, sq_ref):
    xb = x_ref[:]
    y_ref[:] = jax.lax.dot_general(
        xb, w_ref[:], (((1,), (1,)), ((), ())),
        preferred_element_type=jnp.float32)
    sq_ref[:] = jnp.sum(xb * xb, axis=1, keepdims=True)


def _knn_gather_kernel(xb_ref, x_ref, sqb_ref, sqr_ref, y_ref, yb_ref, b_ref,
                       o_ref, idx_vmem, idx_smem, sem, *, k, br, n):
    p = pl.program_id(0)
    g = jax.lax.dot_general(
        xb_ref[:], x_ref[:], (((1,), (1,)), ((), ())),
        preferred_element_type=jnp.float32)
    scores = (sqb_ref[:] + sqr_ref[:]) - 2.0 * g
    col = jax.lax.broadcasted_iota(jnp.int32, (br, n), 1)
    row = jax.lax.broadcasted_iota(jnp.int32, (br, n), 0)
    scores = scores + jnp.where(col == row + p * br, 1e10, 0.0)
    idx_cols = []
    for t in range(k):
        m = jnp.min(scores, axis=1, keepdims=True)
        idxv = jnp.min(jnp.where(scores == m, col, n), axis=1, keepdims=True)
        idx_cols.append(idxv)
        if t < k - 1:
            scores = jnp.where(col == idxv, jnp.float32(3e38), scores)
    idx_vmem[:] = jnp.concatenate(idx_cols, axis=1)
    cp = pltpu.make_async_copy(idx_vmem, idx_smem, sem)
    cp.start()
    cp.wait()

    def body(i, _):
        acc = y_ref[pl.ds(idx_smem[i, 0], 1), :]
        for t in range(1, k):
            acc = jnp.maximum(acc, y_ref[pl.ds(idx_smem[i, t], 1), :])
        o_ref[pl.ds(i, 1), :] = acc
        return 0

    jax.lax.fori_loop(0, br, body, 0)
    o_ref[:] = o_ref[:] - yb_ref[:] + b_ref[:]


def _out_kernel(h_ref, w_ref, b_ref, o_ref):
    o_ref[:] = jax.lax.dot_general(
        h_ref[:], w_ref[:], (((1,), (1,)), ((), ())),
        preferred_element_type=jnp.float32) + b_ref[:]


def _edge_conv(x, W, b, k):
    n, d = x.shape
    h = W.shape[0]
    br = _BR
    y, sq = pl.pallas_call(
        _linear_sq_kernel,
        grid=(n // br,),
        in_specs=[
            pl.BlockSpec((br, d), lambda p: (p, 0)),
            pl.BlockSpec((h, d), lambda p: (0, 0)),
        ],
        out_specs=[
            pl.BlockSpec((br, h), lambda p: (p, 0)),
            pl.BlockSpec((br, 1), lambda p: (p, 0)),
        ],
        out_shape=[
            jax.ShapeDtypeStruct((n, h), jnp.float32),
            jax.ShapeDtypeStruct((n, 1), jnp.float32),
        ],
    )(x, W)
    sqr = sq.reshape(1, n)
    b2d = b.reshape(1, h)
    out = pl.pallas_call(
        functools.partial(_knn_gather_kernel, k=k, br=br, n=n),
        grid=(n // br,),
        in_specs=[
            pl.BlockSpec((br, d), lambda p: (p, 0)),
            pl.BlockSpec((n, d), lambda p: (0, 0)),
            pl.BlockSpec((br, 1), lambda p: (p, 0)),
            pl.BlockSpec((1, n), lambda p: (0, 0)),
            pl.BlockSpec((n, h), lambda p: (0, 0)),
            pl.BlockSpec((br, h), lambda p: (p, 0)),
            pl.BlockSpec((1, h), lambda p: (0, 0)),
        ],
        out_specs=pl.BlockSpec((br, h), lambda p: (p, 0)),
        out_shape=jax.ShapeDtypeStruct((n, h), jnp.float32),
        scratch_shapes=[
            pltpu.VMEM((br, k), jnp.int32),
            pltpu.SMEM((br, k), jnp.int32),
            pltpu.SemaphoreType.DMA,
        ],
    )(x, x, sq, sqr, y, y, b2d)
    return out


def kernel(x, W1, b1, W2, b2, W3, b3):
    h1 = _edge_conv(x, W1, b1, 5)
    h2 = _edge_conv(h1, W2, b2, 10)
    n, h = h2.shape
    c = W3.shape[0]
    br = _BR
    out = pl.pallas_call(
        _out_kernel,
        grid=(n // br,),
        in_specs=[
            pl.BlockSpec((br, h), lambda p: (p, 0)),
            pl.BlockSpec((c, h), lambda p: (0, 0)),
            pl.BlockSpec((1, c), lambda p: (0, 0)),
        ],
        out_specs=pl.BlockSpec((br, c), lambda p: (p, 0)),
        out_shape=jax.ShapeDtypeStruct((n, c), jnp.float32),
    )(h2, W3, b3.reshape(1, c))
    return out


# speedup vs baseline: 4.8776x; 4.8776x over previous
import functools

import jax
import jax.numpy as jnp
from jax.experimental import pallas as pl
from jax.experimental.pallas import tpu as pltpu

_BR = 256


def _sq_kernel(x_ref, sq_ref):
    xb = x_ref[:]
    sq_ref[:] = jnp.sum(xb * xb, axis=1, keepdims=True)


def _knn_conv_kernel(xb_ref, x_ref, sqb_ref, sqr_ref, w_ref, b_ref,
                     o_ref, msg_vmem, idx_vmem, idx_smem, sem, *, k, br, n):
    p = pl.program_id(0)
    g = jax.lax.dot_general(
        xb_ref[:], x_ref[:], (((1,), (1,)), ((), ())),
        preferred_element_type=jnp.float32)
    scores = (sqb_ref[:] + sqr_ref[:]) - 2.0 * g
    col = jax.lax.broadcasted_iota(jnp.int32, (br, n), 1)
    row = jax.lax.broadcasted_iota(jnp.int32, (br, n), 0)
    scores = scores + jnp.where(col == row + p * br, 1e10, 0.0)
    idx_cols = []
    for t in range(k):
        m = jnp.min(scores, axis=1, keepdims=True)
        idxv = jnp.min(jnp.where(scores == m, col, n), axis=1, keepdims=True)
        idx_cols.append(idxv)
        if t < k - 1:
            scores = jnp.where(col == idxv, jnp.float32(3e38), scores)
    idx_vmem[:] = jnp.concatenate(idx_cols, axis=1)
    cp = pltpu.make_async_copy(idx_vmem, idx_smem, sem)
    cp.start()
    cp.wait()

    def body(i, _):
        xi = xb_ref[pl.ds(i, 1), :]
        for t in range(k):
            j = idx_smem[i, t]
            msg_vmem[pl.ds(t * br + i, 1), :] = x_ref[pl.ds(j, 1), :] - xi
        return 0

    jax.lax.fori_loop(0, br, body, 0)

    acc = None
    for t in range(k):
        ht = jax.lax.dot_general(
            msg_vmem[pl.ds(t * br, br), :], w_ref[:],
            (((1,), (1,)), ((), ())), preferred_element_type=jnp.float32)
        acc = ht if acc is None else jnp.maximum(acc, ht)
    o_ref[:] = acc + b_ref[:]


def _out_kernel(h_ref, w_ref, b_ref, o_ref):
    o_ref[:] = jax.lax.dot_general(
        h_ref[:], w_ref[:], (((1,), (1,)), ((), ())),
        preferred_element_type=jnp.float32) + b_ref[:]


def _edge_conv(x, W, b, k):
    n, d = x.shape
    h = W.shape[0]
    br = _BR
    sq = pl.pallas_call(
        _sq_kernel,
        grid=(n // br,),
        in_specs=[pl.BlockSpec((br, d), lambda p: (p, 0))],
        out_specs=pl.BlockSpec((br, 1), lambda p: (p, 0)),
        out_shape=jax.ShapeDtypeStruct((n, 1), jnp.float32),
    )(x)
    sqr = sq.reshape(1, n)
    b2d = b.reshape(1, h)
    out = pl.pallas_call(
        functools.partial(_knn_conv_kernel, k=k, br=br, n=n),
        grid=(n // br,),
        in_specs=[
            pl.BlockSpec((br, d), lambda p: (p, 0)),
            pl.BlockSpec((n, d), lambda p: (0, 0)),
            pl.BlockSpec((br, 1), lambda p: (p, 0)),
            pl.BlockSpec((1, n), lambda p: (0, 0)),
            pl.BlockSpec((h, d), lambda p: (0, 0)),
            pl.BlockSpec((1, h), lambda p: (0, 0)),
        ],
        out_specs=pl.BlockSpec((br, h), lambda p: (p, 0)),
        out_shape=jax.ShapeDtypeStruct((n, h), jnp.float32),
        scratch_shapes=[
            pltpu.VMEM((br * k, d), jnp.float32),
            pltpu.VMEM((br, k), jnp.int32),
            pltpu.SMEM((br, k), jnp.int32),
            pltpu.SemaphoreType.DMA,
        ],
    )(x, x, sq, sqr, W, b2d)
    return out


def kernel(x, W1, b1, W2, b2, W3, b3):
    h1 = _edge_conv(x, W1, b1, 5)
    h2 = _edge_conv(h1, W2, b2, 10)
    n, h = h2.shape
    c = W3.shape[0]
    br = _BR
    out = pl.pallas_call(
        _out_kernel,
        grid=(n // br,),
        in_specs=[
            pl.BlockSpec((br, h), lambda p: (p, 0)),
            pl.BlockSpec((c, h), lambda p: (0, 0)),
            pl.BlockSpec((1, c), lambda p: (0, 0)),
        ],
        out_specs=pl.BlockSpec((br, c), lambda p: (p, 0)),
        out_shape=jax.ShapeDtypeStruct((n, c), jnp.float32),
    )(h2, W3, b3.reshape(1, c))
    return out


# speedup vs baseline: 5.1776x; 1.0615x over previous
import functools

import jax
import jax.numpy as jnp
from jax.experimental import pallas as pl
from jax.experimental.pallas import tpu as pltpu

_BR = 256


def _sq_kernel(x_ref, sq_ref):
    xb = x_ref[:]
    sq_ref[:] = jnp.sum(xb * xb, axis=1, keepdims=True)


def _knn_conv_kernel(xb_ref, x_ref, sqb_ref, sqr_ref, w_ref, b_ref,
                     o_ref, msg_vmem, idx_vmem, idx_smem, sem, *, k, br, n):
    p = pl.program_id(0)
    g = jax.lax.dot_general(
        xb_ref[:], x_ref[:], (((1,), (1,)), ((), ())),
        preferred_element_type=jnp.float32)
    scores = (sqb_ref[:] + sqr_ref[:]) - 2.0 * g
    col = jax.lax.broadcasted_iota(jnp.int32, (br, n), 1)
    row = jax.lax.broadcasted_iota(jnp.int32, (br, n), 0)
    scores = scores + jnp.where(col == row + p * br, 1e10, 0.0)
    idx_cols = []
    for t in range(k):
        idxv = jnp.argmin(scores, axis=1, keepdims=True).astype(jnp.int32)
        idx_cols.append(idxv)
        if t < k - 1:
            scores = jnp.where(col == idxv, jnp.float32(3e38), scores)
    idx_vmem[:] = jnp.concatenate(idx_cols, axis=1)
    cp = pltpu.make_async_copy(idx_vmem, idx_smem, sem)
    cp.start()
    cp.wait()

    def body(i, _):
        xi = xb_ref[pl.ds(i, 1), :]
        for t in range(k):
            j = idx_smem[i, t]
            msg_vmem[pl.ds(t * br + i, 1), :] = x_ref[pl.ds(j, 1), :] - xi
        return 0

    jax.lax.fori_loop(0, br, body, 0)

    acc = None
    for t in range(k):
        ht = jax.lax.dot_general(
            msg_vmem[pl.ds(t * br, br), :], w_ref[:],
            (((1,), (1,)), ((), ())), preferred_element_type=jnp.float32)
        acc = ht if acc is None else jnp.maximum(acc, ht)
    o_ref[:] = acc + b_ref[:]


def _out_kernel(h_ref, w_ref, b_ref, o_ref):
    o_ref[:] = jax.lax.dot_general(
        h_ref[:], w_ref[:], (((1,), (1,)), ((), ())),
        preferred_element_type=jnp.float32) + b_ref[:]


def _edge_conv(x, W, b, k):
    n, d = x.shape
    h = W.shape[0]
    br = _BR
    sq = pl.pallas_call(
        _sq_kernel,
        grid=(n // br,),
        in_specs=[pl.BlockSpec((br, d), lambda p: (p, 0))],
        out_specs=pl.BlockSpec((br, 1), lambda p: (p, 0)),
        out_shape=jax.ShapeDtypeStruct((n, 1), jnp.float32),
    )(x)
    sqr = sq.reshape(1, n)
    b2d = b.reshape(1, h)
    out = pl.pallas_call(
        functools.partial(_knn_conv_kernel, k=k, br=br, n=n),
        grid=(n // br,),
        in_specs=[
            pl.BlockSpec((br, d), lambda p: (p, 0)),
            pl.BlockSpec((n, d), lambda p: (0, 0)),
            pl.BlockSpec((br, 1), lambda p: (p, 0)),
            pl.BlockSpec((1, n), lambda p: (0, 0)),
            pl.BlockSpec((h, d), lambda p: (0, 0)),
            pl.BlockSpec((1, h), lambda p: (0, 0)),
        ],
        out_specs=pl.BlockSpec((br, h), lambda p: (p, 0)),
        out_shape=jax.ShapeDtypeStruct((n, h), jnp.float32),
        scratch_shapes=[
            pltpu.VMEM((br * k, d), jnp.float32),
            pltpu.VMEM((br, k), jnp.int32),
            pltpu.SMEM((br, k), jnp.int32),
            pltpu.SemaphoreType.DMA,
        ],
    )(x, x, sq, sqr, W, b2d)
    return out


def kernel(x, W1, b1, W2, b2, W3, b3):
    h1 = _edge_conv(x, W1, b1, 5)
    h2 = _edge_conv(h1, W2, b2, 10)
    n, h = h2.shape
    c = W3.shape[0]
    br = _BR
    out = pl.pallas_call(
        _out_kernel,
        grid=(n // br,),
        in_specs=[
            pl.BlockSpec((br, h), lambda p: (p, 0)),
            pl.BlockSpec((c, h), lambda p: (0, 0)),
            pl.BlockSpec((1, c), lambda p: (0, 0)),
        ],
        out_specs=pl.BlockSpec((br, c), lambda p: (p, 0)),
        out_shape=jax.ShapeDtypeStruct((n, c), jnp.float32),
    )(h2, W3, b3.reshape(1, c))
    return out
